# baseline (speedup 1.0000x reference)
import jax
import jax.numpy as jnp
from jax import lax
from jax.experimental import pallas as pl
from jax.experimental.pallas import tpu as pltpu
from jax.experimental.pallas import tpu_sc as plsc

R = 1024
N = 100000
NC = 2
NS = 16
SLAB = 128
NSLAB = R // SLAB
NQ = 4
Q = N // NQ
CC = 200
NCHK = Q // CC
BC = 20
NBLK = CC // BC
NGRP = SLAB // 16
NEG_INF = float("-inf")


def _insert(ts, v):
    out = []
    new = v
    for t in ts:
        hi = jnp.maximum(t, new)
        new = jnp.minimum(t, new)
        out.append(hi)
    return out


def _sc_body(predsT_hbm, labels_hbm, out_hbm,
             lab_v, stv, mrg, res_v, buf0, buf1, shared, sem0, sem1):
    cid = lax.axis_index("c")
    sid = lax.axis_index("s")
    slab = cid * (NSLAB // NC) + sid // NQ
    quarter = sid % NQ
    row0 = slab * SLAB
    colq = quarter * Q

    pltpu.sync_copy(labels_hbm.at[pl.ds(row0, SLAB)], lab_v)

    def reset(g, carry_r):
        for j in range(5):
            stv[g * 6 + j] = jnp.full((16,), NEG_INF, jnp.float32)
        stv[g * 6 + 5] = jnp.zeros((16,), jnp.float32)
        return carry_r

    lax.fori_loop(0, NGRP, reset, 0)

    pltpu.async_copy(
        predsT_hbm.at[pl.ds(colq, CC), pl.ds(row0, SLAB)], buf0, sem0)
    pltpu.async_copy(
        predsT_hbm.at[pl.ds(colq + CC, CC), pl.ds(row0, SLAB)], buf1, sem1)

    def chunk_scan(buf, c):
        col0c = colq + c * CC

        def one_group(g, carry_g):
            t0 = stv[g * 6 + 0]
            t1 = stv[g * 6 + 1]
            t2 = stv[g * 6 + 2]
            t3 = stv[g * 6 + 3]
            t4 = stv[g * 6 + 4]
            g16 = g * 16

            def blk(b, ts):
                base = b * BC
                vs = [buf[base + i, pl.ds(g16, 16)] for i in range(BC)]
                accs = [jnp.maximum(vs[2 * i], vs[2 * i + 1])
                        for i in range(BC // 2)]
                acc = accs[0]
                lvl = list(accs)
                while len(lvl) > 1:
                    nxt = [jnp.maximum(lvl[i], lvl[i + 1])
                           for i in range(0, len(lvl) - 1, 2)]
                    if len(lvl) % 2:
                        nxt.append(lvl[-1])
                    lvl = nxt
                acc = lvl[0]
                trig = plsc.all_reduce_population_count(acc > ts[4])[0] > 0

                def slow(ts2):
                    cur = list(ts2)
                    his = [jnp.maximum(vs[2 * i], vs[2 * i + 1])
                           for i in range(BC // 2)]
                    los = [jnp.minimum(vs[2 * i], vs[2 * i + 1])
                           for i in range(BC // 2)]
                    for h in his:
                        cur = _insert(cur, h)
                    lo_lvl = list(los)
                    while len(lo_lvl) > 1:
                        nxt = [jnp.maximum(lo_lvl[i], lo_lvl[i + 1])
                               for i in range(0, len(lo_lvl) - 1, 2)]
                        if len(lo_lvl) % 2:
                            nxt.append(lo_lvl[-1])
                        lo_lvl = nxt
                    lo_trig = plsc.all_reduce_population_count(
                        lo_lvl[0] > cur[4])[0] > 0

                    def lo_slow(ts3):
                        c3 = list(ts3)
                        for lo in los:
                            c3 = _insert(c3, lo)
                        return tuple(c3)

                    return lax.cond(lo_trig, lo_slow, lambda a: a,
                                    tuple(cur))

                return lax.cond(trig, slow, lambda a: a, ts)

            t0, t1, t2, t3, t4 = plsc.parallel_loop(
                0, NBLK, unroll=5, carry=(t0, t1, t2, t3, t4))(blk)
            stv[g * 6 + 0] = t0
            stv[g * 6 + 1] = t1
            stv[g * 6 + 2] = t2
            stv[g * 6 + 3] = t3
            stv[g * 6 + 4] = t4

            labv = lab_v[pl.ds(g16, 16)]
            labv = jnp.minimum(jnp.maximum(labv, 0), N - 1)
            inb = (labv >= col0c) & (labv < col0c + CC)
            li = jnp.where(inb, labv - col0c, 0)
            rows16 = lax.iota(jnp.int32, 16) + g16
            vals = plsc.load_gather(buf, [li, rows16])
            stv[g * 6 + 5] = jnp.where(inb, vals, stv[g * 6 + 5])
            return carry_g

        plsc.parallel_loop(0, NGRP, carry=jnp.int32(0))(one_group)

    def pair(c2, carry_p):
        ca = 2 * c2
        pltpu.make_async_copy(
            predsT_hbm.at[pl.ds(0, CC), pl.ds(0, SLAB)], buf0, sem0).wait()
        chunk_scan(buf0, ca)

        @pl.when(ca + 2 < NCHK)
        def _issue0():
            pltpu.async_copy(
                predsT_hbm.at[pl.ds(colq + (ca + 2) * CC, CC),
                              pl.ds(row0, SLAB)], buf0, sem0)

        pltpu.make_async_copy(
            predsT_hbm.at[pl.ds(0, CC), pl.ds(0, SLAB)], buf1, sem1).wait()
        chunk_scan(buf1, ca + 1)

        @pl.when(ca + 3 < NCHK)
        def _issue1():
            pltpu.async_copy(
                predsT_hbm.at[pl.ds(colq + (ca + 3) * CC, CC),
                              pl.ds(row0, SLAB)], buf1, sem1)

        return carry_p

    lax.fori_loop(0, (NCHK - 1) // 2, pair, 0)

    pltpu.make_async_copy(
        predsT_hbm.at[pl.ds(0, CC), pl.ds(0, SLAB)], buf0, sem0).wait()
    chunk_scan(buf0, NCHK - 1)

    pltpu.sync_copy(stv, shared.at[pl.ds(sid * 6 * NGRP, 6 * NGRP)])
    plsc.subcore_barrier()

    @pl.when(quarter == 0)
    def _do_merge():
        def zero_res(j, carry_z):
            res_v[6, pl.ds(j * 16, 16)] = jnp.zeros((16,), jnp.float32)
            res_v[7, pl.ds(j * 16, 16)] = jnp.zeros((16,), jnp.float32)
            return carry_z

        lax.fori_loop(0, NGRP, zero_res, 0)

        def merge_one(p, carry_m):
            pltpu.sync_copy(shared.at[pl.ds((sid + p) * 6 * NGRP, 6 * NGRP)], mrg)

            def mg(g, carry_g):
                cur = [stv[g * 6 + j] for j in range(5)]
                for j in range(5):
                    cur = _insert(cur, mrg[g * 6 + j])
                for j in range(5):
                    stv[g * 6 + j] = cur[j]
                stv[g * 6 + 5] = stv[g * 6 + 5] + mrg[g * 6 + 5]
                return carry_g

            lax.fori_loop(0, NGRP, mg, 0)
            return carry_m

        lax.fori_loop(1, NQ, merge_one, 0)

        def emit(g, carry_e):
            for j in range(6):
                res_v[j, pl.ds(g * 16, 16)] = stv[g * 6 + j]
            return carry_e

        lax.fori_loop(0, NGRP, emit, 0)
        pltpu.sync_copy(res_v, out_hbm.at[:, pl.ds(row0, SLAB)])


_scan = pl.kernel(
    _sc_body,
    out_type=jax.ShapeDtypeStruct((8, R), jnp.float32),
    mesh=plsc.VectorSubcoreMesh(core_axis_name="c", subcore_axis_name="s"),
    compiler_params=pltpu.CompilerParams(needs_layout_passes=False),
    scratch_types=[
        pltpu.VMEM((SLAB,), jnp.int32),
        pltpu.VMEM((NGRP * 6, 16), jnp.float32),
        pltpu.VMEM((NGRP * 6, 16), jnp.float32),
        pltpu.VMEM((8, SLAB), jnp.float32),
        pltpu.VMEM((CC, SLAB), jnp.float32),
        pltpu.VMEM((CC, SLAB), jnp.float32),
        pltpu.VMEM_SHARED((NS * NGRP * 6, 16), jnp.float32),
        pltpu.SemaphoreType.DMA,
        pltpu.SemaphoreType.DMA,
    ],
)


def _fin_body(top_ref, lab_ref, out_ref):
    arr = top_ref[...]
    labs = lab_ref[...]
    top5 = arr[0:5, :]
    m = jnp.max(top5, axis=0, keepdims=True)
    s = jnp.sum(jnp.exp(top5 - m), axis=0, keepdims=True)
    lse = m + jnp.log(s)
    neg = arr[5:6, :]
    mask = labs != -100
    contrib = jnp.where(mask, lse - neg, 0.0)
    total = jnp.sum(contrib)
    cnt = jnp.sum(mask.astype(jnp.float32))
    out_ref[...] = jnp.reshape(total / cnt, (1, 1))


_finish = pl.pallas_call(
    _fin_body,
    out_shape=jax.ShapeDtypeStruct((1, 1), jnp.float32),
)


def kernel(preds, labels):
    preds = preds.reshape(R, N)
    labels_i = labels.astype(jnp.int32)
    tops = _scan(preds.T, labels_i)
    out = _finish(tops, labels_i.reshape(1, R))
    return jnp.reshape(out, ())


# speedup vs baseline: 1.4222x; 1.4222x over previous
import jax
import jax.numpy as jnp
from jax import lax
from jax.experimental import pallas as pl
from jax.experimental.pallas import tpu as pltpu
from jax.experimental.pallas import tpu_sc as plsc

R = 1024
N = 100000
NC = 2
NS = 16
SLAB = 128
NSLAB = R // SLAB
NQ = 4
Q = N // NQ
CC = 200
NCHK = Q // CC
BC = 20
NBLK = CC // BC
NGRP = SLAB // 16
NEG_INF = float("-inf")


def _insert(ts, v):
    out = []
    new = v
    for t in ts:
        hi = jnp.maximum(t, new)
        new = jnp.minimum(t, new)
        out.append(hi)
    return out


def _sc_body(predsT_hbm, labels_hbm, out_hbm,
             lab_v, stv, mrg, res_v, buf0, buf1, shared, sem0, sem1):
    cid = lax.axis_index("c")
    sid = lax.axis_index("s")
    slab = cid * (NSLAB // NC) + sid // NQ
    quarter = sid % NQ
    row0 = slab * SLAB
    colq = quarter * Q

    pltpu.sync_copy(labels_hbm.at[pl.ds(row0, SLAB)], lab_v)

    def reset(g, carry_r):
        for j in range(5):
            stv[g * 6 + j] = jnp.full((16,), NEG_INF, jnp.float32)
        stv[g * 6 + 5] = jnp.zeros((16,), jnp.float32)
        return carry_r

    lax.fori_loop(0, NGRP, reset, 0)

    pltpu.async_copy(
        predsT_hbm.at[pl.ds(colq, CC), pl.ds(row0, SLAB)], buf0, sem0)
    pltpu.async_copy(
        predsT_hbm.at[pl.ds(colq + CC, CC), pl.ds(row0, SLAB)], buf1, sem1)

    def chunk_scan(buf, c):
        col0c = colq + c * CC

        def one_group(g, carry_g):
            t0 = stv[g * 6 + 0]
            t1 = stv[g * 6 + 1]
            t2 = stv[g * 6 + 2]
            t3 = stv[g * 6 + 3]
            t4 = stv[g * 6 + 4]
            g16 = g * 16

            def blk(b, ts):
                base = b * BC
                vs = [buf[base + i, pl.ds(g16, 16)] for i in range(BC)]
                acc = vs[0]
                accs = list(vs)
                while len(accs) > 1:
                    nxt = [jnp.maximum(accs[i], accs[i + 1])
                           for i in range(0, len(accs) - 1, 2)]
                    if len(accs) % 2:
                        nxt.append(accs[-1])
                    accs = nxt
                acc = accs[0]
                trig = plsc.all_reduce_population_count(acc > ts[4])[0] > 0

                def slow(ts2):
                    cur = list(ts2)
                    for v in vs:
                        cur = _insert(cur, v)
                    return tuple(cur)

                return lax.cond(trig, slow, lambda a: a, ts)

            t0, t1, t2, t3, t4 = plsc.parallel_loop(
                0, NBLK, unroll=10, carry=(t0, t1, t2, t3, t4))(blk)
            stv[g * 6 + 0] = t0
            stv[g * 6 + 1] = t1
            stv[g * 6 + 2] = t2
            stv[g * 6 + 3] = t3
            stv[g * 6 + 4] = t4

            labv = lab_v[pl.ds(g16, 16)]
            labv = jnp.minimum(jnp.maximum(labv, 0), N - 1)
            inb = (labv >= col0c) & (labv < col0c + CC)
            li = jnp.where(inb, labv - col0c, 0)
            rows16 = lax.iota(jnp.int32, 16) + g16
            vals = plsc.load_gather(buf, [li, rows16])
            stv[g * 6 + 5] = jnp.where(inb, vals, stv[g * 6 + 5])
            return carry_g

        plsc.parallel_loop(0, NGRP, carry=jnp.int32(0))(one_group)

    def pair(c2, carry_p):
        ca = 2 * c2
        pltpu.make_async_copy(
            predsT_hbm.at[pl.ds(0, CC), pl.ds(0, SLAB)], buf0, sem0).wait()
        chunk_scan(buf0, ca)

        @pl.when(ca + 2 < NCHK)
        def _issue0():
            pltpu.async_copy(
                predsT_hbm.at[pl.ds(colq + (ca + 2) * CC, CC),
                              pl.ds(row0, SLAB)], buf0, sem0)

        pltpu.make_async_copy(
            predsT_hbm.at[pl.ds(0, CC), pl.ds(0, SLAB)], buf1, sem1).wait()
        chunk_scan(buf1, ca + 1)

        @pl.when(ca + 3 < NCHK)
        def _issue1():
            pltpu.async_copy(
                predsT_hbm.at[pl.ds(colq + (ca + 3) * CC, CC),
                              pl.ds(row0, SLAB)], buf1, sem1)

        return carry_p

    lax.fori_loop(0, (NCHK - 1) // 2, pair, 0)

    pltpu.make_async_copy(
        predsT_hbm.at[pl.ds(0, CC), pl.ds(0, SLAB)], buf0, sem0).wait()
    chunk_scan(buf0, NCHK - 1)

    pltpu.sync_copy(stv, shared.at[pl.ds(sid * 6 * NGRP, 6 * NGRP)])
    plsc.subcore_barrier()

    @pl.when(quarter == 0)
    def _do_merge():
        def zero_res(j, carry_z):
            res_v[6, pl.ds(j * 16, 16)] = jnp.zeros((16,), jnp.float32)
            res_v[7, pl.ds(j * 16, 16)] = jnp.zeros((16,), jnp.float32)
            return carry_z

        lax.fori_loop(0, NGRP, zero_res, 0)

        def merge_one(p, carry_m):
            pltpu.sync_copy(shared.at[pl.ds((sid + p) * 6 * NGRP, 6 * NGRP)], mrg)

            def mg(g, carry_g):
                cur = [stv[g * 6 + j] for j in range(5)]
                for j in range(5):
                    cur = _insert(cur, mrg[g * 6 + j])
                for j in range(5):
                    stv[g * 6 + j] = cur[j]
                stv[g * 6 + 5] = stv[g * 6 + 5] + mrg[g * 6 + 5]
                return carry_g

            lax.fori_loop(0, NGRP, mg, 0)
            return carry_m

        lax.fori_loop(1, NQ, merge_one, 0)

        def emit(g, carry_e):
            for j in range(6):
                res_v[j, pl.ds(g * 16, 16)] = stv[g * 6 + j]
            return carry_e

        lax.fori_loop(0, NGRP, emit, 0)
        pltpu.sync_copy(res_v, out_hbm.at[:, pl.ds(row0, SLAB)])


_scan = pl.kernel(
    _sc_body,
    out_type=jax.ShapeDtypeStruct((8, R), jnp.float32),
    mesh=plsc.VectorSubcoreMesh(core_axis_name="c", subcore_axis_name="s"),
    compiler_params=pltpu.CompilerParams(needs_layout_passes=False),
    scratch_types=[
        pltpu.VMEM((SLAB,), jnp.int32),
        pltpu.VMEM((NGRP * 6, 16), jnp.float32),
        pltpu.VMEM((NGRP * 6, 16), jnp.float32),
        pltpu.VMEM((8, SLAB), jnp.float32),
        pltpu.VMEM((CC, SLAB), jnp.float32),
        pltpu.VMEM((CC, SLAB), jnp.float32),
        pltpu.VMEM_SHARED((NS * NGRP * 6, 16), jnp.float32),
        pltpu.SemaphoreType.DMA,
        pltpu.SemaphoreType.DMA,
    ],
)


def _fin_body(top_ref, lab_ref, out_ref):
    arr = top_ref[...]
    labs = lab_ref[...]
    top5 = arr[0:5, :]
    m = jnp.max(top5, axis=0, keepdims=True)
    s = jnp.sum(jnp.exp(top5 - m), axis=0, keepdims=True)
    lse = m + jnp.log(s)
    neg = arr[5:6, :]
    mask = labs != -100
    contrib = jnp.where(mask, lse - neg, 0.0)
    total = jnp.sum(contrib)
    cnt = jnp.sum(mask.astype(jnp.float32))
    out_ref[...] = jnp.reshape(total / cnt, (1, 1))


_finish = pl.pallas_call(
    _fin_body,
    out_shape=jax.ShapeDtypeStruct((1, 1), jnp.float32),
)


def kernel(preds, labels):
    preds = preds.reshape(R, N)
    labels_i = labels.astype(jnp.int32)
    tops = _scan(preds.T, labels_i)
    out = _finish(tops, labels_i.reshape(1, R))
    return jnp.reshape(out, ())


# speedup vs baseline: 1.6360x; 1.1503x over previous
import jax
import jax.numpy as jnp
from jax import lax
from jax.experimental import pallas as pl
from jax.experimental.pallas import tpu as pltpu
from jax.experimental.pallas import tpu_sc as plsc

R = 1024
N = 100000
NC = 2
NS = 16
SLAB = 128
NSLAB = R // SLAB
NQ = 4
Q = N // NQ
CC = 200
NCHK = Q // CC
BC = 20
NBLK = CC // BC
NGRP = SLAB // 16
NEG_INF = float("-inf")


def _insert(ts, v):
    out = []
    new = v
    for t in ts:
        hi = jnp.maximum(t, new)
        new = jnp.minimum(t, new)
        out.append(hi)
    return out


def _sc_body(predsT_hbm, labels_hbm, out_hbm,
             lab_v, stv, mrg, res_v, buf0, buf1, shared, sem0, sem1):
    cid = lax.axis_index("c")
    sid = lax.axis_index("s")
    slab = cid * (NSLAB // NC) + sid // NQ
    quarter = sid % NQ
    row0 = slab * SLAB
    colq = quarter * Q

    pltpu.sync_copy(labels_hbm.at[pl.ds(row0, SLAB)], lab_v)

    def reset(g, carry_r):
        for j in range(5):
            stv[g * 6 + j] = jnp.full((16,), NEG_INF, jnp.float32)
        stv[g * 6 + 5] = jnp.zeros((16,), jnp.float32)
        return carry_r

    lax.fori_loop(0, NGRP, reset, 0)

    pltpu.async_copy(
        predsT_hbm.at[pl.ds(colq, CC), pl.ds(row0, SLAB)], buf0, sem0)
    pltpu.async_copy(
        predsT_hbm.at[pl.ds(colq + CC, CC), pl.ds(row0, SLAB)], buf1, sem1)

    def chunk_scan(buf, c):
        col0c = colq + c * CC

        def one_group(g, carry_g):
            t0 = stv[g * 6 + 0]
            t1 = stv[g * 6 + 1]
            t2 = stv[g * 6 + 2]
            t3 = stv[g * 6 + 3]
            t4 = stv[g * 6 + 4]
            g16 = g * 16

            def blk(b, ts):
                base = b * BC
                vs = [buf[base + i, pl.ds(g16, 16)] for i in range(BC)]
                acc = vs[0]
                accs = list(vs)
                while len(accs) > 1:
                    nxt = [jnp.maximum(accs[i], accs[i + 1])
                           for i in range(0, len(accs) - 1, 2)]
                    if len(accs) % 2:
                        nxt.append(accs[-1])
                    accs = nxt
                acc = accs[0]
                trig = plsc.all_reduce_population_count(acc > ts[4])[0] > 0

                def slow(ts2):
                    cur = list(ts2)
                    for v in vs:
                        cur = _insert(cur, v)
                    return tuple(cur)

                return lax.cond(trig, slow, lambda a: a, ts)

            t0, t1, t2, t3, t4 = plsc.parallel_loop(
                0, NBLK, unroll=5, carry=(t0, t1, t2, t3, t4))(blk)
            stv[g * 6 + 0] = t0
            stv[g * 6 + 1] = t1
            stv[g * 6 + 2] = t2
            stv[g * 6 + 3] = t3
            stv[g * 6 + 4] = t4

            labv = lab_v[pl.ds(g16, 16)]
            labv = jnp.minimum(jnp.maximum(labv, 0), N - 1)
            inb = (labv >= col0c) & (labv < col0c + CC)
            li = jnp.where(inb, labv - col0c, 0)
            rows16 = lax.iota(jnp.int32, 16) + g16
            vals = plsc.load_gather(buf, [li, rows16])
            stv[g * 6 + 5] = jnp.where(inb, vals, stv[g * 6 + 5])
            return carry_g

        plsc.parallel_loop(0, NGRP, carry=jnp.int32(0))(one_group)

    def pair(c2, carry_p):
        ca = 2 * c2
        pltpu.make_async_copy(
            predsT_hbm.at[pl.ds(0, CC), pl.ds(0, SLAB)], buf0, sem0).wait()
        chunk_scan(buf0, ca)

        @pl.when(ca + 2 < NCHK)
        def _issue0():
            pltpu.async_copy(
                predsT_hbm.at[pl.ds(colq + (ca + 2) * CC, CC),
                              pl.ds(row0, SLAB)], buf0, sem0)

        pltpu.make_async_copy(
            predsT_hbm.at[pl.ds(0, CC), pl.ds(0, SLAB)], buf1, sem1).wait()
        chunk_scan(buf1, ca + 1)

        @pl.when(ca + 3 < NCHK)
        def _issue1():
            pltpu.async_copy(
                predsT_hbm.at[pl.ds(colq + (ca + 3) * CC, CC),
                              pl.ds(row0, SLAB)], buf1, sem1)

        return carry_p

    lax.fori_loop(0, (NCHK - 1) // 2, pair, 0)

    pltpu.make_async_copy(
        predsT_hbm.at[pl.ds(0, CC), pl.ds(0, SLAB)], buf0, sem0).wait()
    chunk_scan(buf0, NCHK - 1)

    pltpu.sync_copy(stv, shared.at[pl.ds(sid * 6 * NGRP, 6 * NGRP)])
    plsc.subcore_barrier()

    @pl.when(quarter == 0)
    def _do_merge():
        def zero_res(j, carry_z):
            res_v[6, pl.ds(j * 16, 16)] = jnp.zeros((16,), jnp.float32)
            res_v[7, pl.ds(j * 16, 16)] = jnp.zeros((16,), jnp.float32)
            return carry_z

        lax.fori_loop(0, NGRP, zero_res, 0)

        def merge_one(p, carry_m):
            pltpu.sync_copy(shared.at[pl.ds((sid + p) * 6 * NGRP, 6 * NGRP)], mrg)

            def mg(g, carry_g):
                cur = [stv[g * 6 + j] for j in range(5)]
                for j in range(5):
                    cur = _insert(cur, mrg[g * 6 + j])
                for j in range(5):
                    stv[g * 6 + j] = cur[j]
                stv[g * 6 + 5] = stv[g * 6 + 5] + mrg[g * 6 + 5]
                return carry_g

            lax.fori_loop(0, NGRP, mg, 0)
            return carry_m

        lax.fori_loop(1, NQ, merge_one, 0)

        def emit(g, carry_e):
            for j in range(6):
                res_v[j, pl.ds(g * 16, 16)] = stv[g * 6 + j]
            return carry_e

        lax.fori_loop(0, NGRP, emit, 0)
        pltpu.sync_copy(res_v, out_hbm.at[:, pl.ds(row0, SLAB)])


_scan = pl.kernel(
    _sc_body,
    out_type=jax.ShapeDtypeStruct((8, R), jnp.float32),
    mesh=plsc.VectorSubcoreMesh(core_axis_name="c", subcore_axis_name="s"),
    compiler_params=pltpu.CompilerParams(needs_layout_passes=False),
    scratch_types=[
        pltpu.VMEM((SLAB,), jnp.int32),
        pltpu.VMEM((NGRP * 6, 16), jnp.float32),
        pltpu.VMEM((NGRP * 6, 16), jnp.float32),
        pltpu.VMEM((8, SLAB), jnp.float32),
        pltpu.VMEM((CC, SLAB), jnp.float32),
        pltpu.VMEM((CC, SLAB), jnp.float32),
        pltpu.VMEM_SHARED((NS * NGRP * 6, 16), jnp.float32),
        pltpu.SemaphoreType.DMA,
        pltpu.SemaphoreType.DMA,
    ],
)


def _fin_body(top_ref, lab_ref, out_ref):
    arr = top_ref[...]
    labs = lab_ref[...]
    top5 = arr[0:5, :]
    m = jnp.max(top5, axis=0, keepdims=True)
    s = jnp.sum(jnp.exp(top5 - m), axis=0, keepdims=True)
    lse = m + jnp.log(s)
    neg = arr[5:6, :]
    mask = labs != -100
    contrib = jnp.where(mask, lse - neg, 0.0)
    total = jnp.sum(contrib)
    cnt = jnp.sum(mask.astype(jnp.float32))
    out_ref[...] = jnp.reshape(total / cnt, (1, 1))


_finish = pl.pallas_call(
    _fin_body,
    out_shape=jax.ShapeDtypeStruct((1, 1), jnp.float32),
)


def kernel(preds, labels):
    preds = preds.reshape(R, N)
    labels_i = labels.astype(jnp.int32)
    tops = _scan(preds.T, labels_i)
    out = _finish(tops, labels_i.reshape(1, R))
    return jnp.reshape(out, ())
